# baseline (speedup 1.0000x reference)
import functools

import jax
import jax.numpy as jnp
from jax import lax
from jax.experimental import pallas as pl
from jax.experimental.pallas import tpu as pltpu
from jax.experimental.pallas import tpu_sc as plsc

K_TOP = 2048
FEATS = 256
N_NODES = 50000
ROWS = 32
ROW_LEN = 2048
N_PAD = ROWS * ROW_LEN

SC_NC = 2
SC_NS = 16
SC_NW = SC_NC * SC_NS
B_PER_W = K_TOP // SC_NW


def _scores_kernel(x_ref, w_ref, o_ref):
    b = pl.program_id(0)
    x = x_ref[...]
    w = w_ref[...]
    s = jnp.dot(x[:, :128], w[:128], preferred_element_type=jnp.float32) + jnp.dot(
        x[:, 128:], w[128:], preferred_element_type=jnp.float32
    )
    s = s.reshape(1, 1, ROW_LEN)
    gidx = b * ROW_LEN + lax.broadcasted_iota(jnp.int32, (1, 1, ROW_LEN), 2)
    o_ref[...] = jnp.where(gidx < N_NODES, s, -jnp.inf)


def _roll1(x, s):
    if s > 0:
        return jnp.concatenate([x[:, -s:], x[:, :-s]], axis=1)
    s = -s
    return jnp.concatenate([x[:, s:], x[:, :s]], axis=1)


def _better(ka, ia, kb, ib):
    return (ka > kb) | ((ka == kb) & (ia < ib))


def _substage(k, i, j, desc):
    l = lax.broadcasted_iota(jnp.int32, k.shape, 1)
    upper = (l & j) != 0
    pk = jnp.where(upper, _roll1(k, j), _roll1(k, -j))
    pi = jnp.where(upper, _roll1(i, j), _roll1(i, -j))
    a_better = _better(k, i, pk, pi)
    keep = (a_better == ~upper) == desc
    return jnp.where(keep, k, pk), jnp.where(keep, i, pi)


def _topk_kernel(s_ref, tv_ref, idx_ref):
    k = s_ref[...].reshape(ROWS, ROW_LEN)
    i = (
        lax.broadcasted_iota(jnp.int32, (ROWS, ROW_LEN), 0) * ROW_LEN
        + lax.broadcasted_iota(jnp.int32, (ROWS, ROW_LEN), 1)
    )
    l = lax.broadcasted_iota(jnp.int32, (ROWS, ROW_LEN), 1)
    row_desc = lax.broadcasted_iota(jnp.int32, (ROWS, ROW_LEN), 0) < ROWS // 2
    level = 2
    while level <= ROW_LEN:
        desc = ((l & level) == 0) == row_desc
        j = level // 2
        while j >= 1:
            k, i = _substage(k, i, j, desc)
            j //= 2
        level *= 2
    rows = ROWS
    while rows > 1:
        half = rows // 2
        ka, ia = k[:half], i[:half]
        kb, ib = k[half:rows], i[half:rows]
        ab = _better(ka, ia, kb, ib)
        k = jnp.where(ab, ka, kb)
        i = jnp.where(ab, ia, ib)
        if half > 1:
            desc = lax.broadcasted_iota(jnp.int32, (half, ROW_LEN), 0) < (half + 1) // 2
        else:
            desc = jnp.ones((1, ROW_LEN), dtype=jnp.bool_)
        j = ROW_LEN // 2
        while j >= 1:
            k, i = _substage(k, i, j, desc)
            j //= 2
        rows = half
    tv_ref[...] = jnp.tanh(k)
    idx_ref[...] = i


@functools.lru_cache(maxsize=1)
def _make_sc_gather():
    @functools.partial(
        pl.kernel,
        mesh=plsc.VectorSubcoreMesh(core_axis_name="c", subcore_axis_name="s"),
        out_type=jax.ShapeDtypeStruct((K_TOP, FEATS), jnp.float32),
        scratch_types=[
            pltpu.VMEM((B_PER_W,), jnp.int32),
            pltpu.VMEM((B_PER_W, FEATS), jnp.float32),
            pltpu.SemaphoreType.DMA,
        ],
    )
    def _sc_gather(table_hbm, idx_hbm, out_hbm, idx_v, rows_v, sem):
        wid = lax.axis_index("s") * SC_NC + lax.axis_index("c")
        base = wid * B_PER_W
        pltpu.sync_copy(idx_hbm.at[pl.ds(base, B_PER_W)], idx_v)
        pltpu.async_copy(table_hbm.at[idx_v], rows_v, sem).wait()
        pltpu.sync_copy(rows_v, out_hbm.at[pl.ds(base, B_PER_W)])

    return _sc_gather


def _scale_t_kernel(g_ref, tv_ref, o_ref):
    g = g_ref[...]
    tv = tv_ref[...]
    o_ref[...] = jnp.transpose(g) * tv


def kernel(node_embs, scorer):
    embs_pad = jnp.pad(node_embs, ((0, N_PAD - N_NODES), (0, 0)))
    raw = pl.pallas_call(
        _scores_kernel,
        grid=(ROWS,),
        in_specs=[
            pl.BlockSpec((ROW_LEN, FEATS), lambda b: (b, 0)),
            pl.BlockSpec((FEATS, 1), lambda b: (0, 0)),
        ],
        out_specs=pl.BlockSpec((1, 1, ROW_LEN), lambda b: (b, 0, 0)),
        out_shape=jax.ShapeDtypeStruct((ROWS, 1, ROW_LEN), jnp.float32),
    )(embs_pad, scorer)
    flat = raw.reshape(N_PAD)[:N_NODES].reshape(N_NODES, 1)
    div = (flat / jnp.linalg.norm(scorer)).reshape(N_NODES)
    scores = (
        jnp.full((N_PAD,), -jnp.inf, dtype=jnp.float32)
        .at[:N_NODES]
        .set(div)
        .reshape(ROWS, 1, ROW_LEN)
    )
    tvals, idx = pl.pallas_call(
        _topk_kernel,
        out_shape=[
            jax.ShapeDtypeStruct((1, K_TOP), jnp.float32),
            jax.ShapeDtypeStruct((1, K_TOP), jnp.int32),
        ],
    )(scores)
    idx_flat = idx.reshape(K_TOP)
    gathered = _make_sc_gather()(node_embs, idx_flat)
    out = pl.pallas_call(
        _scale_t_kernel,
        out_shape=jax.ShapeDtypeStruct((FEATS, K_TOP), jnp.float32),
    )(gathered, tvals)
    return out
